# Initial kernel scaffold; baseline (speedup 1.0000x reference)
import jax
import jax.numpy as jnp
from jax.experimental import pallas as pl


def kernel(n_feats, edge_index, e_types, W1, W1s, b1, W2, W2s, b2, Wmu, Wmus, bmu, Wls, Wlss, bls, Wnt, bnt):
    raise NotImplementedError("write your pallas kernel here")


# speedup vs baseline: 11.7311x; 11.7311x over previous
import functools

import jax
import jax.numpy as jnp
from jax import lax
from jax.experimental import pallas as pl
from jax.experimental.pallas import tpu as pltpu
from jax.experimental.pallas import tpu_sc as plsc

N = 10000
E = 320000
D = 128
R = 4

NC = 2
NS = 16
NW = NC * NS
CH = 128
NCHUNK = 80
SB = 8
EPT = NCHUNK * CH
E_PAD = EPT * NW
ACC_ROWS = N + 8
TRASH = N

BN = 1000
G = N // BN


@functools.partial(
    pl.kernel,
    out_type=jax.ShapeDtypeStruct((NC, N, D), jnp.float32),
    mesh=plsc.VectorSubcoreMesh(core_axis_name="c", subcore_axis_name="s"),
    scratch_types=[
        pltpu.VMEM((SB, CH), jnp.int32),
        pltpu.VMEM((SB, CH), jnp.int32),
        pltpu.VMEM((CH, D), jnp.float32),
        pltpu.VMEM((CH, D), jnp.float32),
        pltpu.VMEM_SHARED((ACC_ROWS, D), jnp.float32),
        pltpu.SemaphoreType.DMA,
        pltpu.SemaphoreType.DMA,
    ],
)
def _sc_scatter(table, gidx, dstx, zeros, out,
                gidx_v, dst_v, rows0, rows1, acc, sg0, sg1):
    c = lax.axis_index("c")
    s = lax.axis_index("s")
    w = c * NS + s

    @pl.when(s == 0)
    def _():
        pltpu.sync_copy(zeros, acc)

    plsc.subcore_barrier()

    def block(b, carry):
        pltpu.sync_copy(gidx.at[w].at[pl.ds(b * SB, SB)], gidx_v)
        pltpu.sync_copy(dstx.at[w].at[pl.ds(b * SB, SB)], dst_v)
        for jp in range(SB // 2):
            j0 = 2 * jp
            j1 = 2 * jp + 1
            h0 = pltpu.async_copy(table.at[gidx_v.at[j0]], rows0, sg0)
            h1 = pltpu.async_copy(table.at[gidx_v.at[j1]], rows1, sg1)
            h0.wait()
            pltpu.sync_copy(rows0, acc.at[dst_v.at[j0]], add=True)
            h1.wait()
            pltpu.sync_copy(rows1, acc.at[dst_v.at[j1]], add=True)
        return carry

    lax.fori_loop(0, NCHUNK // SB, block, 0)

    plsc.subcore_barrier()

    @pl.when(s == 0)
    def _():
        pltpu.sync_copy(acc.at[pl.ds(0, N)], out.at[c])


def _tca_body(x_ref, w_ref, ws_ref, b_ref, xw_ref, sl_ref):
    x = x_ref[...]
    for r in range(R):
        xw_ref[r] = jnp.dot(x, w_ref[r], preferred_element_type=jnp.float32)
    sl_ref[...] = (
        jnp.dot(x, ws_ref[...], preferred_element_type=jnp.float32) + b_ref[...]
    )


_tca = pl.pallas_call(
    _tca_body,
    grid=(G,),
    in_specs=[
        pl.BlockSpec((BN, D), lambda i: (i, 0)),
        pl.BlockSpec((R, D, D), lambda i: (0, 0, 0)),
        pl.BlockSpec((D, D), lambda i: (0, 0)),
        pl.BlockSpec((1, D), lambda i: (0, 0)),
    ],
    out_specs=[
        pl.BlockSpec((R, BN, D), lambda i: (0, i, 0)),
        pl.BlockSpec((BN, D), lambda i: (i, 0)),
    ],
    out_shape=[
        jax.ShapeDtypeStruct((R, N, D), jnp.float32),
        jax.ShapeDtypeStruct((N, D), jnp.float32),
    ],
)


def _tcb_body(p_ref, sl_ref, w_ref, ws_ref, b_ref, xw_ref, sl2_ref):
    h = jnp.maximum(p_ref[0] + p_ref[1] + sl_ref[...], 0.0)
    for r in range(R):
        xw_ref[r] = jnp.dot(h, w_ref[r], preferred_element_type=jnp.float32)
    sl2_ref[...] = (
        jnp.dot(h, ws_ref[...], preferred_element_type=jnp.float32) + b_ref[...]
    )


_tcb = pl.pallas_call(
    _tcb_body,
    grid=(G,),
    in_specs=[
        pl.BlockSpec((NC, BN, D), lambda i: (0, i, 0)),
        pl.BlockSpec((BN, D), lambda i: (i, 0)),
        pl.BlockSpec((R, D, D), lambda i: (0, 0, 0)),
        pl.BlockSpec((D, D), lambda i: (0, 0)),
        pl.BlockSpec((1, D), lambda i: (0, 0)),
    ],
    out_specs=[
        pl.BlockSpec((R, BN, D), lambda i: (0, i, 0)),
        pl.BlockSpec((BN, D), lambda i: (i, 0)),
    ],
    out_shape=[
        jax.ShapeDtypeStruct((R, N, D), jnp.float32),
        jax.ShapeDtypeStruct((N, D), jnp.float32),
    ],
)


def _tcc_body(p_ref, sl_ref, wmu_ref, wmus_ref, bmu_ref,
              wls_ref, wlss_ref, bls_ref,
              xwmu_ref, slmu_ref, xwls_ref, slls_ref):
    h = jnp.maximum(p_ref[0] + p_ref[1] + sl_ref[...], 0.0)
    for r in range(R):
        xwmu_ref[r] = jnp.dot(h, wmu_ref[r], preferred_element_type=jnp.float32)
        xwls_ref[r] = jnp.dot(h, wls_ref[r], preferred_element_type=jnp.float32)
    slmu_ref[...] = (
        jnp.dot(h, wmus_ref[...], preferred_element_type=jnp.float32)
        + bmu_ref[...]
    )
    slls_ref[...] = (
        jnp.dot(h, wlss_ref[...], preferred_element_type=jnp.float32)
        + bls_ref[...]
    )


_tcc = pl.pallas_call(
    _tcc_body,
    grid=(G,),
    in_specs=[
        pl.BlockSpec((NC, BN, D), lambda i: (0, i, 0)),
        pl.BlockSpec((BN, D), lambda i: (i, 0)),
        pl.BlockSpec((R, D, D), lambda i: (0, 0, 0)),
        pl.BlockSpec((D, D), lambda i: (0, 0)),
        pl.BlockSpec((1, D), lambda i: (0, 0)),
        pl.BlockSpec((R, D, D), lambda i: (0, 0, 0)),
        pl.BlockSpec((D, D), lambda i: (0, 0)),
        pl.BlockSpec((1, D), lambda i: (0, 0)),
    ],
    out_specs=[
        pl.BlockSpec((R, BN, D), lambda i: (0, i, 0)),
        pl.BlockSpec((BN, D), lambda i: (i, 0)),
        pl.BlockSpec((R, BN, D), lambda i: (0, i, 0)),
        pl.BlockSpec((BN, D), lambda i: (i, 0)),
    ],
    out_shape=[
        jax.ShapeDtypeStruct((R, N, D), jnp.float32),
        jax.ShapeDtypeStruct((N, D), jnp.float32),
        jax.ShapeDtypeStruct((R, N, D), jnp.float32),
        jax.ShapeDtypeStruct((N, D), jnp.float32),
    ],
)


def _tcd_body(pmu_ref, slmu_ref, pls_ref, slls_ref, wnt_ref, bnt_ref, eps_ref,
              nt_ref, gz_ref, mu_ref, ls_ref):
    i = pl.program_id(0)
    mu = pmu_ref[0] + pmu_ref[1] + slmu_ref[...]
    ls = pls_ref[0] + pls_ref[1] + slls_ref[...]
    z = mu + eps_ref[...] * jnp.exp(ls)
    nt_ref[...] = (
        jnp.dot(z, wnt_ref[...], preferred_element_type=jnp.float32)
        + bnt_ref[...]
    )
    mu_ref[...] = mu
    ls_ref[...] = ls

    @pl.when(i == 0)
    def _():
        gz_ref[...] = jnp.zeros_like(gz_ref)

    gz_ref[...] += jnp.sum(z, axis=0, keepdims=True)

    @pl.when(i == G - 1)
    def _():
        gz_ref[...] = gz_ref[...] * (1.0 / N)


_tcd = pl.pallas_call(
    _tcd_body,
    grid=(G,),
    in_specs=[
        pl.BlockSpec((NC, BN, D), lambda i: (0, i, 0)),
        pl.BlockSpec((BN, D), lambda i: (i, 0)),
        pl.BlockSpec((NC, BN, D), lambda i: (0, i, 0)),
        pl.BlockSpec((BN, D), lambda i: (i, 0)),
        pl.BlockSpec((D, D), lambda i: (0, 0)),
        pl.BlockSpec((1, D), lambda i: (0, 0)),
        pl.BlockSpec((BN, D), lambda i: (i, 0)),
    ],
    out_specs=[
        pl.BlockSpec((BN, D), lambda i: (i, 0)),
        pl.BlockSpec((1, D), lambda i: (0, 0)),
        pl.BlockSpec((BN, D), lambda i: (i, 0)),
        pl.BlockSpec((BN, D), lambda i: (i, 0)),
    ],
    out_shape=[
        jax.ShapeDtypeStruct((N, D), jnp.float32),
        jax.ShapeDtypeStruct((1, D), jnp.float32),
        jax.ShapeDtypeStruct((N, D), jnp.float32),
        jax.ShapeDtypeStruct((N, D), jnp.float32),
    ],
)


def kernel(n_feats, edge_index, e_types, W1, W1s, b1, W2, W2s, b2,
           Wmu, Wmus, bmu, Wls, Wlss, bls, Wnt, bnt):
    src = edge_index[0]
    dst = edge_index[1]
    gidx = (e_types * N + src).astype(jnp.int32)
    pad = E_PAD - E
    gidx_p = jnp.concatenate(
        [gidx, jnp.zeros((pad,), jnp.int32)]).reshape(NW, NCHUNK, CH)
    dst_p = jnp.concatenate(
        [dst.astype(jnp.int32), jnp.full((pad,), TRASH, jnp.int32)]
    ).reshape(NW, NCHUNK, CH)
    zeros = jnp.zeros((ACC_ROWS, D), jnp.float32)

    b1r = b1.reshape(1, D)
    b2r = b2.reshape(1, D)
    bmur = bmu.reshape(1, D)
    blsr = bls.reshape(1, D)
    bntr = bnt.reshape(1, D)

    xw1, sl1 = _tca(n_feats, W1, W1s, b1r)
    p1 = _sc_scatter(xw1.reshape(R * N, D), gidx_p, dst_p, zeros)
    xw2, sl2 = _tcb(p1, sl1, W2, W2s, b2r)
    p2 = _sc_scatter(xw2.reshape(R * N, D), gidx_p, dst_p, zeros)
    xwmu, slmu, xwls, slls = _tcc(p2, sl2, Wmu, Wmus, bmur, Wls, Wlss, blsr)
    pmu = _sc_scatter(xwmu.reshape(R * N, D), gidx_p, dst_p, zeros)
    pls = _sc_scatter(xwls.reshape(R * N, D), gidx_p, dst_p, zeros)
    eps = jax.random.normal(jax.random.key(42), (N, D), dtype=jnp.float32)
    node_types, gz, mu, logstd = _tcd(pmu, slmu, pls, slls, Wnt, bntr, eps)
    return (node_types, gz, mu, logstd)
